# baseline (speedup 1.0000x reference)
import jax
import jax.numpy as jnp
from jax import lax
from jax.experimental import pallas as pl
from jax.experimental.pallas import tpu as pltpu
from jax.experimental.pallas import tpu_sc as plsc

_B, _C, _N = 16, 256, 16384
_K = 4
_ROWS = _B * _C
_SC_ROWS = 1280
_TC_ROWS = _ROWS - _SC_ROWS
_NC, _NS = 2, 16
_NW = _NC * _NS
_SC_IN = _SC_ROWS * _N
_IN_PER_W = _SC_IN // _NW
_CH_IN = 32768
_CH_OUT = _CH_IN // _K
_N_CHUNKS = _IN_PER_W // _CH_IN
_VREGS = _CH_IN // 16
_TC_BLK = 128


def _sc_pool_body(x_hbm, out_hbm, xv0, xv1, ov0, ov1, is0, is1, os0, os1):
    wid = lax.axis_index("s") * _NC + lax.axis_index("c")
    in_base = wid * _IN_PER_W
    out_base = wid * (_IN_PER_W // _K)
    lane = lax.broadcasted_iota(jnp.int32, (16,), 0)
    perm1 = lane ^ 1
    perm2 = lane ^ 2
    mask4 = (lane & 3) == 0
    xvs, ovs = (xv0, xv1), (ov0, ov1)
    isems, osems = (is0, is1), (os0, os1)

    pltpu.async_copy(x_hbm.at[pl.ds(in_base, _CH_IN)], xv0, is0)

    def pair_body(c0, carry):
        for b in (0, 1):
            c = 2 * c0 + b
            pltpu.make_async_copy(
                x_hbm.at[pl.ds(0, _CH_IN)], xvs[b], isems[b]).wait()

            @pl.when(c + 1 < _N_CHUNKS)
            def _():
                pltpu.async_copy(
                    x_hbm.at[pl.ds(in_base + (c + 1) * _CH_IN, _CH_IN)],
                    xvs[1 - b], isems[1 - b])

            @pl.when(c0 >= 1)
            def _():
                pltpu.make_async_copy(
                    ovs[b].at[pl.ds(0, _CH_OUT)],
                    out_hbm.at[pl.ds(0, _CH_OUT)], osems[b]).wait()

            x_v, out_v = xvs[b], ovs[b]

            @plsc.parallel_loop(0, _VREGS, unroll=8)
            def _(i):
                v = x_v[pl.ds(i * 16, 16)]
                s1 = v + jnp.take_along_axis(v, perm1, axis=0)
                s2 = s1 + jnp.take_along_axis(s1, perm2, axis=0)
                plsc.store_compressed(
                    out_v.at[pl.ds(i * 4, 16)], s2 * 0.25, mask=mask4)

            pltpu.async_copy(
                out_v.at[pl.ds(0, _CH_OUT)],
                out_hbm.at[pl.ds(out_base + c * _CH_OUT, _CH_OUT)],
                osems[b])
        return carry

    lax.fori_loop(0, _N_CHUNKS // 2, pair_body, 0)
    for b in (0, 1):
        pltpu.make_async_copy(
            ovs[b].at[pl.ds(0, _CH_OUT)],
            out_hbm.at[pl.ds(0, _CH_OUT)], osems[b]).wait()


def _sc_pool(x_flat):
    mesh = plsc.VectorSubcoreMesh(core_axis_name="c", subcore_axis_name="s")
    return pl.kernel(
        _sc_pool_body,
        out_type=jax.ShapeDtypeStruct((_SC_IN // _K,), jnp.float32),
        mesh=mesh,
        scratch_types=[
            pltpu.VMEM((_CH_IN,), jnp.float32),
            pltpu.VMEM((_CH_IN,), jnp.float32),
            pltpu.VMEM((_CH_OUT + 16,), jnp.float32),
            pltpu.VMEM((_CH_OUT + 16,), jnp.float32),
            pltpu.SemaphoreType.DMA,
            pltpu.SemaphoreType.DMA,
            pltpu.SemaphoreType.DMA,
            pltpu.SemaphoreType.DMA,
        ],
        compiler_params=pltpu.CompilerParams(needs_layout_passes=False),
    )(x_flat)


def _tc_pool_kernel(x_ref, o_ref):
    blk = x_ref[...]
    xb = blk.reshape(_TC_BLK * 128, 128)
    l = lax.broadcasted_iota(jnp.int32, (128, 32), 0)
    m = lax.broadcasted_iota(jnp.int32, (128, 32), 1)
    p = jnp.where((l >> 2) == m, 0.25, 0.0).astype(jnp.float32)
    y = lax.dot_general(xb, p, (((1,), (0,)), ((), ())),
                        preferred_element_type=jnp.float32)
    o_ref[...] = y.reshape(_TC_BLK, 128, 32)


def _tc_pool(x3):
    base = _SC_ROWS // _TC_BLK
    return pl.pallas_call(
        _tc_pool_kernel,
        grid=(_TC_ROWS // _TC_BLK,),
        in_specs=[pl.BlockSpec((_TC_BLK, 128, 128),
                               lambda i: (base + i, 0, 0))],
        out_specs=pl.BlockSpec((_TC_BLK, 128, 32), lambda i: (i, 0, 0)),
        out_shape=jax.ShapeDtypeStruct((_TC_ROWS, 128, 32), jnp.float32),
    )(x3)


def kernel(x):
    x3 = x.reshape(_ROWS, 128, 128)
    sc_out = _sc_pool(x.reshape(_ROWS * _N))
    tc_out = _tc_pool(x3)
    out = jnp.concatenate(
        [sc_out.reshape(_SC_ROWS, _N // _K),
         tc_out.reshape(_TC_ROWS, _N // _K)], axis=0)
    return out.reshape(_B, _C, _N // _K)
